# baseline (speedup 1.0000x reference)
import jax
import jax.numpy as jnp
from jax.experimental import pallas as pl

NEG_SLOPE = 0.2
N = 10000
BLK = 1000


def _combine1_body(agg_ref, z_ref, wl_ref, bl_ref, wr_ref, o_ref):
    acc = jnp.dot(agg_ref[...], wl_ref[...], preferred_element_type=jnp.float32)
    acc += jnp.dot(z_ref[...], wr_ref[...], preferred_element_type=jnp.float32)
    acc += bl_ref[...]
    o_ref[...] = jnp.maximum(acc, 0.0)


def _combine2_body(agg_ref, z_ref, wl_ref, bl_ref, wr_ref, wo_ref, bo_ref, o_ref):
    acc = jnp.dot(agg_ref[...], wl_ref[...], preferred_element_type=jnp.float32)
    acc += jnp.dot(z_ref[...], wr_ref[...], preferred_element_type=jnp.float32)
    acc += bl_ref[...]
    h = jnp.maximum(acc, 0.0)
    o_ref[...] = jnp.dot(h, wo_ref[...], preferred_element_type=jnp.float32) + bo_ref[...]


def _combine1(agg, z, wl, bl, wr):
    grid = (N // BLK,)
    return pl.pallas_call(
        _combine1_body,
        grid=grid,
        in_specs=[
            pl.BlockSpec((BLK, 128), lambda i: (i, 0)),
            pl.BlockSpec((BLK, 128), lambda i: (i, 0)),
            pl.BlockSpec((128, 128), lambda i: (0, 0)),
            pl.BlockSpec((1, 128), lambda i: (0, 0)),
            pl.BlockSpec((128, 128), lambda i: (0, 0)),
        ],
        out_specs=pl.BlockSpec((BLK, 128), lambda i: (i, 0)),
        out_shape=jax.ShapeDtypeStruct((N, 128), jnp.float32),
    )(agg, z, wl, bl.reshape(1, 128), wr)


def _combine2(agg, z, wl, bl, wr, wo, bo):
    grid = (N // BLK,)
    return pl.pallas_call(
        _combine2_body,
        grid=grid,
        in_specs=[
            pl.BlockSpec((BLK, 128), lambda i: (i, 0)),
            pl.BlockSpec((BLK, 128), lambda i: (i, 0)),
            pl.BlockSpec((128, 128), lambda i: (0, 0)),
            pl.BlockSpec((1, 128), lambda i: (0, 0)),
            pl.BlockSpec((128, 128), lambda i: (0, 0)),
            pl.BlockSpec((128, 1), lambda i: (0, 0)),
            pl.BlockSpec((1, 1), lambda i: (0, 0)),
        ],
        out_specs=pl.BlockSpec((BLK, 1), lambda i: (i, 0)),
        out_shape=jax.ShapeDtypeStruct((N, 1), jnp.float32),
    )(agg, z, wl, bl.reshape(1, 128), wr, wo, bo.reshape(1, 1))


def _sparse_phase(z, src, dst, att_src_w, att_dst_w):
    a_i = (z @ att_src_w)[:, 0]
    a_j = (z @ att_dst_w)[:, 0]
    alpha = a_i[dst] + a_j[src]
    alpha = jnp.where(alpha > 0, alpha, NEG_SLOPE * alpha)
    amax = jax.ops.segment_max(alpha, dst, num_segments=N)
    amax = jnp.where(jnp.isfinite(amax), amax, 0.0)
    alpha = jnp.exp(alpha - amax[dst])
    denom = jax.ops.segment_sum(alpha, dst, num_segments=N)
    alpha = alpha / (denom[dst] + 1e-16)
    msg = alpha[:, None] * z[src]
    agg = jax.ops.segment_max(msg, dst, num_segments=N)
    return jnp.where(jnp.isfinite(agg), agg, 0.0)


def kernel(x, edge_index, lin_l0_w, lin_l0_b, lin_r0_w, att_src0, att_dst0,
           lin_l1_w, lin_l1_b, lin_r1_w, att_src1, att_dst1, out_w, out_b):
    src = edge_index[0]
    dst = edge_index[1]
    agg0 = _sparse_phase(x, src, dst, att_src0, att_dst0)
    h = _combine1(agg0, x, lin_l0_w, lin_l0_b, lin_r0_w)
    agg1 = _sparse_phase(h, src, dst, att_src1, att_dst1)
    return _combine2(agg1, h, lin_l1_w, lin_l1_b, lin_r1_w, out_w, out_b)


# speedup vs baseline: 7.5130x; 7.5130x over previous
import functools

import jax
import jax.numpy as jnp
from jax import lax
from jax.experimental import pallas as pl
from jax.experimental.pallas import tpu as pltpu
from jax.experimental.pallas import tpu_sc as plsc

N = 10000
E = 320000
D = 128
NEG_SLOPE = 0.2

NW = 32
NPW = 313
NPAD = NW * NPW
ECA = E // NW
CH = 8000
NCH = E // CH
FCAP = 8064
GB = 128

_BLK = 1000


def _proj_body(z_ref, ws_ref, wd_ref, as_ref, ad_ref):
    z = z_ref[...]
    as_ref[...] = jnp.dot(z, ws_ref[...], preferred_element_type=jnp.float32)
    ad_ref[...] = jnp.dot(z, wd_ref[...], preferred_element_type=jnp.float32)


def _proj(z, att_s, att_d):
    return pl.pallas_call(
        _proj_body,
        grid=(N // _BLK,),
        in_specs=[
            pl.BlockSpec((_BLK, D), lambda i: (i, 0)),
            pl.BlockSpec((D, 1), lambda i: (0, 0)),
            pl.BlockSpec((D, 1), lambda i: (0, 0)),
        ],
        out_specs=[
            pl.BlockSpec((_BLK, 1), lambda i: (i, 0)),
            pl.BlockSpec((_BLK, 1), lambda i: (i, 0)),
        ],
        out_shape=[
            jax.ShapeDtypeStruct((N, 1), jnp.float32),
            jax.ShapeDtypeStruct((N, 1), jnp.float32),
        ],
    )(z, att_s, att_d)


def _combine1_body(agg_ref, z_ref, wl_ref, bl_ref, wr_ref, ws_ref, wd_ref,
                   h_ref, as_ref, ad_ref):
    acc = jnp.dot(agg_ref[...], wl_ref[...], preferred_element_type=jnp.float32)
    acc += jnp.dot(z_ref[...], wr_ref[...], preferred_element_type=jnp.float32)
    acc += bl_ref[...]
    h = jnp.maximum(acc, 0.0)
    h_ref[...] = h
    as_ref[...] = jnp.dot(h, ws_ref[...], preferred_element_type=jnp.float32)
    ad_ref[...] = jnp.dot(h, wd_ref[...], preferred_element_type=jnp.float32)


def _combine1(agg, z, wl, bl, wr, att_s, att_d):
    return pl.pallas_call(
        _combine1_body,
        grid=(N // _BLK,),
        in_specs=[
            pl.BlockSpec((_BLK, D), lambda i: (i, 0)),
            pl.BlockSpec((_BLK, D), lambda i: (i, 0)),
            pl.BlockSpec((D, D), lambda i: (0, 0)),
            pl.BlockSpec((1, D), lambda i: (0, 0)),
            pl.BlockSpec((D, D), lambda i: (0, 0)),
            pl.BlockSpec((D, 1), lambda i: (0, 0)),
            pl.BlockSpec((D, 1), lambda i: (0, 0)),
        ],
        out_specs=[
            pl.BlockSpec((_BLK, D), lambda i: (i, 0)),
            pl.BlockSpec((_BLK, 1), lambda i: (i, 0)),
            pl.BlockSpec((_BLK, 1), lambda i: (i, 0)),
        ],
        out_shape=[
            jax.ShapeDtypeStruct((N, D), jnp.float32),
            jax.ShapeDtypeStruct((N, 1), jnp.float32),
            jax.ShapeDtypeStruct((N, 1), jnp.float32),
        ],
    )(agg, z, wl, bl.reshape(1, D), wr, att_s, att_d)


def _combine2_body(agg_ref, z_ref, wl_ref, bl_ref, wr_ref, wo_ref, bo_ref, o_ref):
    acc = jnp.dot(agg_ref[...], wl_ref[...], preferred_element_type=jnp.float32)
    acc += jnp.dot(z_ref[...], wr_ref[...], preferred_element_type=jnp.float32)
    acc += bl_ref[...]
    h = jnp.maximum(acc, 0.0)
    o_ref[...] = jnp.dot(h, wo_ref[...], preferred_element_type=jnp.float32) + bo_ref[...]


def _combine2(agg, z, wl, bl, wr, wo, bo):
    return pl.pallas_call(
        _combine2_body,
        grid=(N // _BLK,),
        in_specs=[
            pl.BlockSpec((_BLK, D), lambda i: (i, 0)),
            pl.BlockSpec((_BLK, D), lambda i: (i, 0)),
            pl.BlockSpec((D, D), lambda i: (0, 0)),
            pl.BlockSpec((1, D), lambda i: (0, 0)),
            pl.BlockSpec((D, D), lambda i: (0, 0)),
            pl.BlockSpec((D, 1), lambda i: (0, 0)),
            pl.BlockSpec((1, 1), lambda i: (0, 0)),
        ],
        out_specs=pl.BlockSpec((_BLK, 1), lambda i: (i, 0)),
        out_shape=jax.ShapeDtypeStruct((N, 1), jnp.float32),
    )(agg, z, wl, bl.reshape(1, D), wr, wo, bo.reshape(1, 1))


@functools.lru_cache(maxsize=None)
def _make_edge_exp_sc():
    mesh = plsc.VectorSubcoreMesh(core_axis_name="c", subcore_axis_name="s")
    return functools.partial(
        pl.kernel,
        mesh=mesh,
        compiler_params=pltpu.CompilerParams(needs_layout_passes=False),
        out_type=jax.ShapeDtypeStruct((E,), jnp.float32),
        scratch_types=[
            pltpu.VMEM((N,), jnp.float32),
            pltpu.VMEM((N,), jnp.float32),
            pltpu.VMEM((ECA,), jnp.int32),
            pltpu.VMEM((ECA,), jnp.int32),
            pltpu.VMEM((ECA,), jnp.float32),
        ],
    )(_edge_exp_sc_body)


def _edge_exp_sc_body(dst_hbm, src_hbm, adst_hbm, asrc_hbm, e_hbm,
                      adst_v, asrc_v, dv, sv, ev):
    wid = lax.axis_index("s") * 2 + lax.axis_index("c")
    base = wid * ECA
    pltpu.sync_copy(adst_hbm, adst_v)
    pltpu.sync_copy(asrc_hbm, asrc_v)
    pltpu.sync_copy(dst_hbm.at[pl.ds(base, ECA)], dv)
    pltpu.sync_copy(src_hbm.at[pl.ds(base, ECA)], sv)

    def body(i, _):
        o = i * 16
        di = dv[pl.ds(o, 16)]
        si = sv[pl.ds(o, 16)]
        a = plsc.load_gather(adst_v, [di]) + plsc.load_gather(asrc_v, [si])
        a = jnp.where(a > 0.0, a, NEG_SLOPE * a)
        a = jnp.minimum(a, 60.0)
        ev[pl.ds(o, 16)] = jnp.exp(a)
        return _

    lax.fori_loop(0, ECA // 16, body, None)
    pltpu.sync_copy(ev, e_hbm.at[pl.ds(base, ECA)])


@functools.lru_cache(maxsize=None)
def _make_agg_sc():
    mesh = plsc.VectorSubcoreMesh(core_axis_name="c", subcore_axis_name="s")
    return functools.partial(
        pl.kernel,
        mesh=mesh,
        compiler_params=pltpu.CompilerParams(needs_layout_passes=False),
        out_type=jax.ShapeDtypeStruct((NPAD * D,), jnp.float32),
        scratch_types=[
            pltpu.VMEM((CH,), jnp.int32),
            pltpu.VMEM((CH,), jnp.int32),
            pltpu.VMEM((CH,), jnp.float32),
            pltpu.VMEM((FCAP,), jnp.int32),
            pltpu.VMEM((FCAP,), jnp.int32),
            pltpu.VMEM((FCAP,), jnp.float32),
            pltpu.VMEM((GB, D), jnp.float32),
            pltpu.VMEM((NPW * D,), jnp.float32),
            pltpu.SMEM((NPW + 7,), jnp.float32),
            pltpu.SemaphoreType.DMA,
        ],
    )(_agg_sc_body)


def _agg_sc_body(dst_hbm, src_hbm, e_hbm, z_hbm, agg_hbm,
                 dvb, svb, evb, fsrc, fdst, fe, zbuf, um, den, sem):
    wid = lax.axis_index("s") * 2 + lax.axis_index("c")
    lo = wid * NPW
    hi = lo + NPW

    def init_um(i, _):
        um[pl.ds(i * 16, 16)] = jnp.full((16,), -3e38, jnp.float32)
        return _
    lax.fori_loop(0, NPW * D // 16, init_um, None)

    def init_den(i, _):
        den[i] = 0.0
        return _
    lax.fori_loop(0, NPW, init_den, None)

    def init_fsrc(i, _):
        fsrc[pl.ds(i * 16, 16)] = lax.iota(jnp.int32, 16) + i * 16
        return _
    lax.fori_loop(0, FCAP // 16, init_fsrc, None)

    def chunk(g, _):
        cbase = g * CH
        pltpu.sync_copy(dst_hbm.at[pl.ds(cbase, CH)], dvb)
        pltpu.sync_copy(src_hbm.at[pl.ds(cbase, CH)], svb)
        pltpu.sync_copy(e_hbm.at[pl.ds(cbase, CH)], evb)

        def filt(v, cnt):
            o = v * 16
            d = dvb[pl.ds(o, 16)]
            m = (d >= lo) & (d < hi)
            mi = m.astype(jnp.int32)
            pos = cnt + plsc.cumsum(mi) - 1
            plsc.store_scatter(fsrc, [pos], svb[pl.ds(o, 16)], mask=m)
            plsc.store_scatter(fdst, [pos], d - lo, mask=m)
            plsc.store_scatter(fe, [pos], evb[pl.ds(o, 16)], mask=m)
            return cnt + jnp.sum(mi)

        cnt = lax.fori_loop(0, CH // 16, filt, jnp.int32(0))
        nb = (cnt + (GB - 1)) // GB

        def batch(j, _):
            pltpu.async_copy(
                z_hbm.at[fsrc.at[pl.ds(j * GB, GB)]], zbuf, sem).wait()
            kend = jnp.minimum(cnt, (j + 1) * GB)

            def edge(k, _):
                dstl = fdst[pl.ds(k, 16)][0]
                ee = fe[pl.ds(k, 16)][0]
                r = k - j * GB
                eb = jnp.full((16,), ee, jnp.float32)
                ubase = dstl * D
                for jj in range(D // 16):
                    zv = zbuf[r, pl.ds(jj * 16, 16)]
                    uo = ubase + jj * 16
                    um[pl.ds(uo, 16)] = jnp.maximum(um[pl.ds(uo, 16)], zv * eb)
                den[dstl] = den[dstl] + ee
                return _

            lax.fori_loop(j * GB, kend, edge, None)
            return _

        lax.fori_loop(0, nb, batch, None)
        return _

    lax.fori_loop(0, NCH, chunk, None)

    def norm(i, _):
        dd = den[i]
        db = jnp.full((16,), dd, jnp.float32)
        rb = jnp.where(db == 0.0, jnp.zeros((16,), jnp.float32),
                       jnp.ones((16,), jnp.float32) / (db + 1e-16))
        for jj in range(D // 16):
            uo = i * D + jj * 16
            um[pl.ds(uo, 16)] = um[pl.ds(uo, 16)] * rb
        return _

    lax.fori_loop(0, NPW, norm, None)
    pltpu.sync_copy(um, agg_hbm.at[pl.ds(lo * D, NPW * D)])


def kernel(x, edge_index, lin_l0_w, lin_l0_b, lin_r0_w, att_src0, att_dst0,
           lin_l1_w, lin_l1_b, lin_r1_w, att_src1, att_dst1, out_w, out_b):
    src = edge_index[0]
    dst = edge_index[1]

    edge_exp_sc = _make_edge_exp_sc()
    agg_sc = _make_agg_sc()
    as0, ad0 = _proj(x, att_src0, att_dst0)
    e0 = edge_exp_sc(dst, src, as0.reshape(N), ad0.reshape(N))
    agg0 = agg_sc(dst, src, e0, x).reshape(NPAD, D)[:N]
    h, as1, ad1 = _combine1(agg0, x, lin_l0_w, lin_l0_b, lin_r0_w,
                            att_src1, att_dst1)
    e1 = edge_exp_sc(dst, src, as1.reshape(N), ad1.reshape(N))
    agg1 = agg_sc(dst, src, e1, h).reshape(NPAD, D)[:N]
    return _combine2(agg1, h, lin_l1_w, lin_l1_b, lin_r1_w, out_w, out_b)


# speedup vs baseline: 13.5167x; 1.7991x over previous
import functools

import jax
import jax.numpy as jnp
from jax import lax
from jax.experimental import pallas as pl
from jax.experimental.pallas import tpu as pltpu
from jax.experimental.pallas import tpu_sc as plsc

N = 10000
E = 320000
D = 128
NEG_SLOPE = 0.2

NW = 32
NPW = 313
NPAD = NW * NPW
FC = 8000
NCHF = E // FC
DR = 4096
FBUF = DR + FC + 16
EROW = E + DR
FC2 = 4096
GB = 128
PADROW = NPW * D

_BLK = 1000


def _proj_body(z_ref, ws_ref, wd_ref, as_ref, ad_ref):
    z = z_ref[...]
    as_ref[...] = jnp.dot(z, ws_ref[...], preferred_element_type=jnp.float32)
    ad_ref[...] = jnp.dot(z, wd_ref[...], preferred_element_type=jnp.float32)


def _proj(z, att_s, att_d):
    return pl.pallas_call(
        _proj_body,
        grid=(N // _BLK,),
        in_specs=[
            pl.BlockSpec((_BLK, D), lambda i: (i, 0)),
            pl.BlockSpec((D, 1), lambda i: (0, 0)),
            pl.BlockSpec((D, 1), lambda i: (0, 0)),
        ],
        out_specs=[
            pl.BlockSpec((_BLK, 1), lambda i: (i, 0)),
            pl.BlockSpec((_BLK, 1), lambda i: (i, 0)),
        ],
        out_shape=[
            jax.ShapeDtypeStruct((N, 1), jnp.float32),
            jax.ShapeDtypeStruct((N, 1), jnp.float32),
        ],
    )(z, att_s, att_d)


def _combine1_body(agg_ref, z_ref, wl_ref, bl_ref, wr_ref, ws_ref, wd_ref,
                   h_ref, as_ref, ad_ref):
    acc = jnp.dot(agg_ref[...], wl_ref[...], preferred_element_type=jnp.float32)
    acc += jnp.dot(z_ref[...], wr_ref[...], preferred_element_type=jnp.float32)
    acc += bl_ref[...]
    h = jnp.maximum(acc, 0.0)
    h_ref[...] = h
    as_ref[...] = jnp.dot(h, ws_ref[...], preferred_element_type=jnp.float32)
    ad_ref[...] = jnp.dot(h, wd_ref[...], preferred_element_type=jnp.float32)


def _combine1(agg, z, wl, bl, wr, att_s, att_d):
    return pl.pallas_call(
        _combine1_body,
        grid=(N // _BLK,),
        in_specs=[
            pl.BlockSpec((_BLK, D), lambda i: (i, 0)),
            pl.BlockSpec((_BLK, D), lambda i: (i, 0)),
            pl.BlockSpec((D, D), lambda i: (0, 0)),
            pl.BlockSpec((1, D), lambda i: (0, 0)),
            pl.BlockSpec((D, D), lambda i: (0, 0)),
            pl.BlockSpec((D, 1), lambda i: (0, 0)),
            pl.BlockSpec((D, 1), lambda i: (0, 0)),
        ],
        out_specs=[
            pl.BlockSpec((_BLK, D), lambda i: (i, 0)),
            pl.BlockSpec((_BLK, 1), lambda i: (i, 0)),
            pl.BlockSpec((_BLK, 1), lambda i: (i, 0)),
        ],
        out_shape=[
            jax.ShapeDtypeStruct((N, D), jnp.float32),
            jax.ShapeDtypeStruct((N, 1), jnp.float32),
            jax.ShapeDtypeStruct((N, 1), jnp.float32),
        ],
    )(agg, z, wl, bl.reshape(1, D), wr, att_s, att_d)


def _combine2_body(agg_ref, z_ref, wl_ref, bl_ref, wr_ref, wo_ref, bo_ref, o_ref):
    acc = jnp.dot(agg_ref[...], wl_ref[...], preferred_element_type=jnp.float32)
    acc += jnp.dot(z_ref[...], wr_ref[...], preferred_element_type=jnp.float32)
    acc += bl_ref[...]
    h = jnp.maximum(acc, 0.0)
    o_ref[...] = jnp.dot(h, wo_ref[...], preferred_element_type=jnp.float32) + bo_ref[...]


def _combine2(agg, z, wl, bl, wr, wo, bo):
    return pl.pallas_call(
        _combine2_body,
        grid=(N // _BLK,),
        in_specs=[
            pl.BlockSpec((_BLK, D), lambda i: (i, 0)),
            pl.BlockSpec((_BLK, D), lambda i: (i, 0)),
            pl.BlockSpec((D, D), lambda i: (0, 0)),
            pl.BlockSpec((1, D), lambda i: (0, 0)),
            pl.BlockSpec((D, D), lambda i: (0, 0)),
            pl.BlockSpec((D, 1), lambda i: (0, 0)),
            pl.BlockSpec((1, 1), lambda i: (0, 0)),
        ],
        out_specs=pl.BlockSpec((_BLK, 1), lambda i: (i, 0)),
        out_shape=jax.ShapeDtypeStruct((N, 1), jnp.float32),
    )(agg, z, wl, bl.reshape(1, D), wr, wo, bo.reshape(1, 1))


@functools.lru_cache(maxsize=None)
def _make_filter_sc():
    mesh = plsc.VectorSubcoreMesh(core_axis_name="c", subcore_axis_name="s")
    return functools.partial(
        pl.kernel,
        mesh=mesh,
        compiler_params=pltpu.CompilerParams(needs_layout_passes=False),
        out_type=[
            jax.ShapeDtypeStruct((NW * EROW,), jnp.int32),
            jax.ShapeDtypeStruct((NW * EROW,), jnp.int32),
            jax.ShapeDtypeStruct((NW * 16,), jnp.int32),
        ],
        scratch_types=[
            pltpu.VMEM((FC,), jnp.int32),
            pltpu.VMEM((FC,), jnp.int32),
            pltpu.VMEM((FBUF,), jnp.int32),
            pltpu.VMEM((FBUF,), jnp.int32),
            pltpu.VMEM((16,), jnp.int32),
        ],
    )(_filter_sc_body)


def _filter_sc_body(dst_hbm, src_hbm, fd_hbm, fs_hbm, cnt_hbm,
                    dvb, svb, bdst, bsrc, cbuf):
    wid = lax.axis_index("s") * 2 + lax.axis_index("c")
    lo = wid * NPW
    hi = lo + NPW
    wbase = wid * EROW

    def drain(bc, ndr):
        def do(args):
            bc, ndr = args
            off = wbase + ndr * DR
            pltpu.sync_copy(bdst.at[pl.ds(0, DR)], fd_hbm.at[pl.ds(off, DR)])
            pltpu.sync_copy(bsrc.at[pl.ds(0, DR)], fs_hbm.at[pl.ds(off, DR)])
            nmv = (bc - DR + 15) // 16

            def mv(i, _):
                o = i * 16
                bdst[pl.ds(o, 16)] = bdst[pl.ds(DR + o, 16)]
                bsrc[pl.ds(o, 16)] = bsrc[pl.ds(DR + o, 16)]
                return _

            lax.fori_loop(0, nmv, mv, None)
            return (bc - DR, ndr + 1)

        return lax.cond(bc >= DR, do, lambda a: a, (bc, ndr))

    def chunk(g, carry):
        bc, ndr = carry
        cbase = g * FC
        pltpu.sync_copy(dst_hbm.at[pl.ds(cbase, FC)], dvb)
        pltpu.sync_copy(src_hbm.at[pl.ds(cbase, FC)], svb)

        def filt(v, bc):
            o = v * 16
            d = dvb[pl.ds(o, 16)]
            m = (d >= lo) & (d < hi)
            plsc.store_compressed(bdst.at[pl.ds(bc, 16)], d, mask=m)
            plsc.store_compressed(bsrc.at[pl.ds(bc, 16)], svb[pl.ds(o, 16)], mask=m)
            pc = plsc.all_reduce_population_count(m)
            return bc + pc[0]

        bc = lax.fori_loop(0, FC // 16, filt, bc)
        bc, ndr = drain(bc, ndr)
        bc, ndr = drain(bc, ndr)
        return (bc, ndr)

    bc, ndr = lax.fori_loop(0, NCHF, chunk, (jnp.int32(0), jnp.int32(0)))
    off = wbase + ndr * DR
    pltpu.sync_copy(bdst.at[pl.ds(0, DR)], fd_hbm.at[pl.ds(off, DR)])
    pltpu.sync_copy(bsrc.at[pl.ds(0, DR)], fs_hbm.at[pl.ds(off, DR)])
    total = ndr * DR + bc
    cbuf[pl.ds(0, 16)] = jnp.full((16,), 0, jnp.int32) + total
    pltpu.sync_copy(cbuf, cnt_hbm.at[pl.ds(wid * 16, 16)])


@functools.lru_cache(maxsize=None)
def _make_agg_sc():
    mesh = plsc.VectorSubcoreMesh(core_axis_name="c", subcore_axis_name="s")
    return functools.partial(
        pl.kernel,
        mesh=mesh,
        compiler_params=pltpu.CompilerParams(needs_layout_passes=False),
        out_type=jax.ShapeDtypeStruct((NPAD * D,), jnp.float32),
        scratch_types=[
            pltpu.VMEM((N,), jnp.float32),
            pltpu.VMEM((N,), jnp.float32),
            pltpu.VMEM((FC2,), jnp.int32),
            pltpu.VMEM((FC2,), jnp.int32),
            pltpu.VMEM((FC2,), jnp.float32),
            pltpu.VMEM((FC2,), jnp.int32),
            pltpu.VMEM((2 * GB, D), jnp.float32),
            pltpu.VMEM(((NPW + 1) * D,), jnp.float32),
            pltpu.VMEM((16,), jnp.int32),
            pltpu.SMEM((NPW + 7,), jnp.float32),
            pltpu.SemaphoreType.DMA,
        ],
    )(_agg_sc_body)


def _agg_sc_body(fd_hbm, fs_hbm, cnt_hbm, adst_hbm, asrc_hbm, z_hbm, agg_hbm,
                 adst_v, asrc_v, fdc, fsc, feb, obuf, zbuf, um, cbuf, den, sem):
    wid = lax.axis_index("s") * 2 + lax.axis_index("c")
    lo = wid * NPW
    wbase = wid * EROW

    pltpu.sync_copy(adst_hbm, adst_v)
    pltpu.sync_copy(asrc_hbm, asrc_v)
    pltpu.sync_copy(cnt_hbm.at[pl.ds(wid * 16, 16)], cbuf)
    cnt = cbuf[pl.ds(0, 16)][0]

    def init_um(i, _):
        um[pl.ds(i * 16, 16)] = jnp.full((16,), -3e38, jnp.float32)
        return _
    lax.fori_loop(0, (NPW + 1) * D // 16, init_um, None)

    def init_den(i, _):
        den[i] = 0.0
        return _
    lax.fori_loop(0, NPW, init_den, None)

    def chunk(c, _):
        pltpu.sync_copy(fd_hbm.at[pl.ds(wbase + c * FC2, FC2)], fdc)
        pltpu.sync_copy(fs_hbm.at[pl.ds(wbase + c * FC2, FC2)], fsc)
        ne = jnp.minimum(FC2, cnt - c * FC2)
        nv = (ne + 15) // 16
        nb = (ne + GB - 1) // GB

        def escore(v, _):
            o = v * 16
            d = fdc[pl.ds(o, 16)]
            s = fsc[pl.ds(o, 16)]
            valid = (lax.iota(jnp.int32, 16) + o) < ne
            safe = lax.iota(jnp.int32, 16) + (v & 511) * 16
            s = jnp.where(valid, s, safe)
            fsc[pl.ds(o, 16)] = s
            a = plsc.load_gather(adst_v, [jnp.where(valid, d, 0)]) + \
                plsc.load_gather(asrc_v, [s])
            a = jnp.where(a > 0.0, a, NEG_SLOPE * a)
            a = jnp.minimum(a, 60.0)
            feb[pl.ds(o, 16)] = jnp.where(valid, jnp.exp(a), 0.0)
            obuf[pl.ds(o, 16)] = jnp.where(valid, (d - lo) * D, PADROW)
            return _

        lax.fori_loop(0, nv, escore, None)

        def sanitize(v, _):
            o = v * 16
            fsc[pl.ds(o, 16)] = lax.iota(jnp.int32, 16) + (v & 511) * 16
            return _

        lax.fori_loop(nv, nb * (GB // 16), sanitize, None)

        def fire(j, slot):
            pltpu.async_copy(
                z_hbm.at[fsc.at[pl.ds(j * GB, GB)]],
                zbuf.at[pl.ds(slot * GB, GB)], sem)

        @pl.when(nb > 0)
        def _prologue():
            fire(jnp.int32(0), jnp.int32(0))

        def batch(j, _):
            @pl.when(j + 1 < nb)
            def _next():
                fire(j + 1, (j + 1) % 2)

            pltpu.make_async_copy(
                z_hbm.at[fsc.at[pl.ds(0, GB)]],
                zbuf.at[pl.ds(0, GB)], sem).wait()

            slotbase = (j % 2) * GB - j * GB
            kend = jnp.minimum(ne, (j + 1) * GB)
            ng = (kend - j * GB + 15) // 16

            def group(g, _):
                o = j * GB + g * 16
                ov = obuf[pl.ds(o, 16)]
                ev = feb[pl.ds(o, 16)]
                for jj in range(16):
                    ub = ov[jj]
                    eb = jnp.full((16,), ev[jj], jnp.float32)
                    ri = slotbase + o + jj
                    dl = jnp.right_shift(ub, 7)
                    for dd in range(D // 16):
                        zv = zbuf[ri, pl.ds(dd * 16, 16)]
                        uo = ub + dd * 16
                        um[pl.ds(uo, 16)] = jnp.maximum(um[pl.ds(uo, 16)], zv * eb)
                    den[dl] = den[dl] + ev[jj]
                return _

            lax.fori_loop(0, ng, group, None)
            return _

        lax.fori_loop(0, nb, batch, None)
        return _

    nc = (cnt + FC2 - 1) // FC2
    lax.fori_loop(0, nc, chunk, None)

    def norm(i, _):
        dd = den[i]
        db = jnp.full((16,), dd, jnp.float32)
        rb = jnp.where(db == 0.0, jnp.zeros((16,), jnp.float32),
                       jnp.ones((16,), jnp.float32) / (db + 1e-16))
        for jj in range(D // 16):
            uo = i * D + jj * 16
            um[pl.ds(uo, 16)] = um[pl.ds(uo, 16)] * rb
        return _

    lax.fori_loop(0, NPW, norm, None)
    pltpu.sync_copy(um.at[pl.ds(0, NPW * D)], agg_hbm.at[pl.ds(lo * D, NPW * D)])


def kernel(x, edge_index, lin_l0_w, lin_l0_b, lin_r0_w, att_src0, att_dst0,
           lin_l1_w, lin_l1_b, lin_r1_w, att_src1, att_dst1, out_w, out_b):
    src = edge_index[0]
    dst = edge_index[1]

    filter_sc = _make_filter_sc()
    agg_sc = _make_agg_sc()

    fd, fs, cnts = filter_sc(dst, src)
    as0, ad0 = _proj(x, att_src0, att_dst0)
    agg0 = agg_sc(fd, fs, cnts, as0.reshape(N), ad0.reshape(N), x)
    agg0 = agg0.reshape(NPAD, D)[:N]
    h, as1, ad1 = _combine1(agg0, x, lin_l0_w, lin_l0_b, lin_r0_w,
                            att_src1, att_dst1)
    agg1 = agg_sc(fd, fs, cnts, as1.reshape(N), ad1.reshape(N), h)
    agg1 = agg1.reshape(NPAD, D)[:N]
    return _combine2(agg1, h, lin_l1_w, lin_l1_b, lin_r1_w, out_w, out_b)


# speedup vs baseline: 25.6124x; 1.8949x over previous
import functools

import jax
import jax.numpy as jnp
from jax import lax
from jax.experimental import pallas as pl
from jax.experimental.pallas import tpu as pltpu
from jax.experimental.pallas import tpu_sc as plsc

N = 10000
E = 320000
D = 128
NEG_SLOPE = 0.2

NW = 32
NPW = 313
NPAD = NW * NPW
FC = 8000
NCHF = E // FC
DR = 4096
FBUF = DR + FC + 16
EROW = E + DR
FC2 = 4096
GB = 128
PADROW = NPW * D

_BLK = 1000


def _proj_body(z_ref, ws_ref, wd_ref, as_ref, ad_ref):
    z = z_ref[...]
    as_ref[...] = jnp.dot(z, ws_ref[...], preferred_element_type=jnp.float32)
    ad_ref[...] = jnp.dot(z, wd_ref[...], preferred_element_type=jnp.float32)


def _proj(z, att_s, att_d):
    return pl.pallas_call(
        _proj_body,
        grid=(N // _BLK,),
        in_specs=[
            pl.BlockSpec((_BLK, D), lambda i: (i, 0)),
            pl.BlockSpec((D, 1), lambda i: (0, 0)),
            pl.BlockSpec((D, 1), lambda i: (0, 0)),
        ],
        out_specs=[
            pl.BlockSpec((_BLK, 1), lambda i: (i, 0)),
            pl.BlockSpec((_BLK, 1), lambda i: (i, 0)),
        ],
        out_shape=[
            jax.ShapeDtypeStruct((N, 1), jnp.float32),
            jax.ShapeDtypeStruct((N, 1), jnp.float32),
        ],
    )(z, att_s, att_d)


def _combine1_body(agg_ref, z_ref, wl_ref, bl_ref, wr_ref, ws_ref, wd_ref,
                   h_ref, as_ref, ad_ref):
    acc = jnp.dot(agg_ref[...], wl_ref[...], preferred_element_type=jnp.float32)
    acc += jnp.dot(z_ref[...], wr_ref[...], preferred_element_type=jnp.float32)
    acc += bl_ref[...]
    h = jnp.maximum(acc, 0.0)
    h_ref[...] = h
    as_ref[...] = jnp.dot(h, ws_ref[...], preferred_element_type=jnp.float32)
    ad_ref[...] = jnp.dot(h, wd_ref[...], preferred_element_type=jnp.float32)


def _combine1(agg, z, wl, bl, wr, att_s, att_d):
    return pl.pallas_call(
        _combine1_body,
        grid=(N // _BLK,),
        in_specs=[
            pl.BlockSpec((_BLK, D), lambda i: (i, 0)),
            pl.BlockSpec((_BLK, D), lambda i: (i, 0)),
            pl.BlockSpec((D, D), lambda i: (0, 0)),
            pl.BlockSpec((1, D), lambda i: (0, 0)),
            pl.BlockSpec((D, D), lambda i: (0, 0)),
            pl.BlockSpec((D, 1), lambda i: (0, 0)),
            pl.BlockSpec((D, 1), lambda i: (0, 0)),
        ],
        out_specs=[
            pl.BlockSpec((_BLK, D), lambda i: (i, 0)),
            pl.BlockSpec((_BLK, 1), lambda i: (i, 0)),
            pl.BlockSpec((_BLK, 1), lambda i: (i, 0)),
        ],
        out_shape=[
            jax.ShapeDtypeStruct((N, D), jnp.float32),
            jax.ShapeDtypeStruct((N, 1), jnp.float32),
            jax.ShapeDtypeStruct((N, 1), jnp.float32),
        ],
    )(agg, z, wl, bl.reshape(1, D), wr, att_s, att_d)


def _combine2_body(agg_ref, z_ref, wl_ref, bl_ref, wr_ref, wo_ref, bo_ref, o_ref):
    acc = jnp.dot(agg_ref[...], wl_ref[...], preferred_element_type=jnp.float32)
    acc += jnp.dot(z_ref[...], wr_ref[...], preferred_element_type=jnp.float32)
    acc += bl_ref[...]
    h = jnp.maximum(acc, 0.0)
    o_ref[...] = jnp.dot(h, wo_ref[...], preferred_element_type=jnp.float32) + bo_ref[...]


def _combine2(agg, z, wl, bl, wr, wo, bo):
    return pl.pallas_call(
        _combine2_body,
        grid=(N // _BLK,),
        in_specs=[
            pl.BlockSpec((_BLK, D), lambda i: (i, 0)),
            pl.BlockSpec((_BLK, D), lambda i: (i, 0)),
            pl.BlockSpec((D, D), lambda i: (0, 0)),
            pl.BlockSpec((1, D), lambda i: (0, 0)),
            pl.BlockSpec((D, D), lambda i: (0, 0)),
            pl.BlockSpec((D, 1), lambda i: (0, 0)),
            pl.BlockSpec((1, 1), lambda i: (0, 0)),
        ],
        out_specs=pl.BlockSpec((_BLK, 1), lambda i: (i, 0)),
        out_shape=jax.ShapeDtypeStruct((N, 1), jnp.float32),
    )(agg, z, wl, bl.reshape(1, D), wr, wo, bo.reshape(1, 1))


@functools.lru_cache(maxsize=None)
def _make_filter_sc():
    mesh = plsc.VectorSubcoreMesh(core_axis_name="c", subcore_axis_name="s")
    return functools.partial(
        pl.kernel,
        mesh=mesh,
        compiler_params=pltpu.CompilerParams(needs_layout_passes=False),
        out_type=[
            jax.ShapeDtypeStruct((NW * EROW,), jnp.int32),
            jax.ShapeDtypeStruct((NW * EROW,), jnp.int32),
            jax.ShapeDtypeStruct((NW * 16,), jnp.int32),
        ],
        scratch_types=[
            pltpu.VMEM((2 * FC,), jnp.int32),
            pltpu.VMEM((2 * FC,), jnp.int32),
            pltpu.VMEM((FBUF,), jnp.int32),
            pltpu.VMEM((FBUF,), jnp.int32),
            pltpu.VMEM((16,), jnp.int32),
            pltpu.SemaphoreType.DMA,
        ],
    )(_filter_sc_body)


def _filter_sc_body(dst_hbm, src_hbm, fd_hbm, fs_hbm, cnt_hbm,
                    dvb, svb, bdst, bsrc, cbuf, semf):
    wid = lax.axis_index("s") * 2 + lax.axis_index("c")
    lo = wid * NPW
    hi = lo + NPW
    wbase = wid * EROW

    def fire(g, slot):
        pltpu.async_copy(dst_hbm.at[pl.ds(g * FC, FC)],
                         dvb.at[pl.ds(slot * FC, FC)], semf)
        pltpu.async_copy(src_hbm.at[pl.ds(g * FC, FC)],
                         svb.at[pl.ds(slot * FC, FC)], semf)

    fire(jnp.int32(0), jnp.int32(0))

    def drain(bc, ndr):
        def do(args):
            bc, ndr = args
            off = wbase + ndr * DR
            pltpu.sync_copy(bdst.at[pl.ds(0, DR)], fd_hbm.at[pl.ds(off, DR)])
            pltpu.sync_copy(bsrc.at[pl.ds(0, DR)], fs_hbm.at[pl.ds(off, DR)])
            nmv = (bc - DR + 15) // 16

            def mv(i, _):
                o = i * 16
                bdst[pl.ds(o, 16)] = bdst[pl.ds(DR + o, 16)]
                bsrc[pl.ds(o, 16)] = bsrc[pl.ds(DR + o, 16)]
                return _

            lax.fori_loop(0, nmv, mv, None)
            return (bc - DR, ndr + 1)

        return lax.cond(bc >= DR, do, lambda a: a, (bc, ndr))

    def chunk(g, carry):
        bc, ndr = carry

        @pl.when(g + 1 < NCHF)
        def _next():
            fire(g + 1, (g + 1) % 2)

        pltpu.make_async_copy(dst_hbm.at[pl.ds(0, FC)],
                              dvb.at[pl.ds(0, FC)], semf).wait()
        pltpu.make_async_copy(src_hbm.at[pl.ds(0, FC)],
                              svb.at[pl.ds(0, FC)], semf).wait()
        sbase = (g % 2) * FC

        def filt(v, bc):
            o = sbase + v * 16
            d = dvb[pl.ds(o, 16)]
            m = (d >= lo) & (d < hi)
            plsc.store_compressed(bdst.at[pl.ds(bc, 16)], d, mask=m)
            plsc.store_compressed(bsrc.at[pl.ds(bc, 16)], svb[pl.ds(o, 16)], mask=m)
            pc = plsc.all_reduce_population_count(m)
            return bc + pc[0]

        bc = lax.fori_loop(0, FC // 16, filt, bc)
        bc, ndr = drain(bc, ndr)
        bc, ndr = drain(bc, ndr)
        return (bc, ndr)

    bc, ndr = lax.fori_loop(0, NCHF, chunk, (jnp.int32(0), jnp.int32(0)))
    off = wbase + ndr * DR
    pltpu.sync_copy(bdst.at[pl.ds(0, DR)], fd_hbm.at[pl.ds(off, DR)])
    pltpu.sync_copy(bsrc.at[pl.ds(0, DR)], fs_hbm.at[pl.ds(off, DR)])
    total = ndr * DR + bc
    cbuf[pl.ds(0, 16)] = jnp.full((16,), 0, jnp.int32) + total
    pltpu.sync_copy(cbuf, cnt_hbm.at[pl.ds(wid * 16, 16)])


@functools.lru_cache(maxsize=None)
def _make_agg_sc():
    mesh = plsc.VectorSubcoreMesh(core_axis_name="c", subcore_axis_name="s")
    return functools.partial(
        pl.kernel,
        mesh=mesh,
        compiler_params=pltpu.CompilerParams(needs_layout_passes=False),
        out_type=jax.ShapeDtypeStruct((NPAD * D,), jnp.float32),
        scratch_types=[
            pltpu.VMEM((N,), jnp.float32),
            pltpu.VMEM((N,), jnp.float32),
            pltpu.VMEM((FC2,), jnp.int32),
            pltpu.VMEM((FC2,), jnp.int32),
            pltpu.VMEM((FC2,), jnp.float32),
            pltpu.VMEM((FC2,), jnp.int32),
            pltpu.VMEM((2 * GB, D), jnp.float32),
            pltpu.VMEM(((NPW + 1) * D,), jnp.float32),
            pltpu.VMEM((16,), jnp.int32),
            pltpu.SMEM((NPW + 7,), jnp.float32),
            pltpu.SemaphoreType.DMA,
        ],
    )(_agg_sc_body)


def _agg_sc_body(fd_hbm, fs_hbm, cnt_hbm, adst_hbm, asrc_hbm, z_hbm, agg_hbm,
                 adst_v, asrc_v, fdc, fsc, feb, obuf, zbuf, um, cbuf, den, sem):
    wid = lax.axis_index("s") * 2 + lax.axis_index("c")
    lo = wid * NPW
    wbase = wid * EROW

    pltpu.sync_copy(adst_hbm, adst_v)
    pltpu.sync_copy(asrc_hbm, asrc_v)
    pltpu.sync_copy(cnt_hbm.at[pl.ds(wid * 16, 16)], cbuf)
    cnt = cbuf[pl.ds(0, 16)][0]

    def init_um(i, _):
        um[pl.ds(i * 16, 16)] = jnp.full((16,), -3e38, jnp.float32)
        return _
    lax.fori_loop(0, (NPW + 1) * D // 16, init_um, None)

    def init_den(i, _):
        den[i] = 0.0
        return _
    lax.fori_loop(0, NPW, init_den, None)

    def chunk(c, _):
        pltpu.sync_copy(fd_hbm.at[pl.ds(wbase + c * FC2, FC2)], fdc)
        pltpu.sync_copy(fs_hbm.at[pl.ds(wbase + c * FC2, FC2)], fsc)
        ne = jnp.minimum(FC2, cnt - c * FC2)
        nv = (ne + 15) // 16
        nb = (ne + GB - 1) // GB

        def escore(v, _):
            o = v * 16
            d = fdc[pl.ds(o, 16)]
            s = fsc[pl.ds(o, 16)]
            valid = (lax.iota(jnp.int32, 16) + o) < ne
            safe = lax.iota(jnp.int32, 16) + (v & 511) * 16
            s = jnp.where(valid, s, safe)
            fsc[pl.ds(o, 16)] = s
            a = plsc.load_gather(adst_v, [jnp.where(valid, d, 0)]) + \
                plsc.load_gather(asrc_v, [s])
            a = jnp.where(a > 0.0, a, NEG_SLOPE * a)
            a = jnp.minimum(a, 60.0)
            feb[pl.ds(o, 16)] = jnp.where(valid, jnp.exp(a), 0.0)
            obuf[pl.ds(o, 16)] = jnp.where(valid, (d - lo) * D, PADROW)
            return _

        lax.fori_loop(0, nv, escore, None)

        def sanitize(v, _):
            o = v * 16
            fsc[pl.ds(o, 16)] = lax.iota(jnp.int32, 16) + (v & 511) * 16
            return _

        lax.fori_loop(nv, nb * (GB // 16), sanitize, None)

        def fire(j, slot):
            pltpu.async_copy(
                z_hbm.at[fsc.at[pl.ds(j * GB, GB)]],
                zbuf.at[pl.ds(slot * GB, GB)], sem)

        @pl.when(nb > 0)
        def _prologue():
            fire(jnp.int32(0), jnp.int32(0))

        def batch(j, _):
            @pl.when(j + 1 < nb)
            def _next():
                fire(j + 1, (j + 1) % 2)

            pltpu.make_async_copy(
                z_hbm.at[fsc.at[pl.ds(0, GB)]],
                zbuf.at[pl.ds(0, GB)], sem).wait()

            slotbase = (j % 2) * GB - j * GB
            kend = jnp.minimum(ne, (j + 1) * GB)
            ng = (kend - j * GB + 15) // 16

            def group(g, _):
                o = j * GB + g * 16
                ov = obuf[pl.ds(o, 16)]
                ev = feb[pl.ds(o, 16)]
                for jj in range(16):
                    ub = ov[jj]
                    eb = jnp.full((16,), ev[jj], jnp.float32)
                    ri = slotbase + o + jj
                    dl = jnp.right_shift(ub, 7)
                    zvs = [zbuf[ri, pl.ds(dd * 16, 16)] for dd in range(D // 16)]
                    accs = [um[pl.ds(ub + dd * 16, 16)] for dd in range(D // 16)]
                    for dd in range(D // 16):
                        um[pl.ds(ub + dd * 16, 16)] = jnp.maximum(
                            accs[dd], zvs[dd] * eb)
                    den[dl] = den[dl] + ev[jj]
                return _

            lax.fori_loop(0, ng, group, None)
            return _

        lax.fori_loop(0, nb, batch, None)
        return _

    nc = (cnt + FC2 - 1) // FC2
    lax.fori_loop(0, nc, chunk, None)

    def norm(i, _):
        dd = den[i]
        db = jnp.full((16,), dd, jnp.float32)
        rb = jnp.where(db == 0.0, jnp.zeros((16,), jnp.float32),
                       jnp.ones((16,), jnp.float32) / (db + 1e-16))
        for jj in range(D // 16):
            uo = i * D + jj * 16
            um[pl.ds(uo, 16)] = um[pl.ds(uo, 16)] * rb
        return _

    lax.fori_loop(0, NPW, norm, None)
    pltpu.sync_copy(um.at[pl.ds(0, NPW * D)], agg_hbm.at[pl.ds(lo * D, NPW * D)])


def kernel(x, edge_index, lin_l0_w, lin_l0_b, lin_r0_w, att_src0, att_dst0,
           lin_l1_w, lin_l1_b, lin_r1_w, att_src1, att_dst1, out_w, out_b):
    src = edge_index[0]
    dst = edge_index[1]

    filter_sc = _make_filter_sc()
    agg_sc = _make_agg_sc()

    fd, fs, cnts = filter_sc(dst, src)
    as0, ad0 = _proj(x, att_src0, att_dst0)
    agg0 = agg_sc(fd, fs, cnts, as0.reshape(N), ad0.reshape(N), x)
    agg0 = agg0.reshape(NPAD, D)[:N]
    h, as1, ad1 = _combine1(agg0, x, lin_l0_w, lin_l0_b, lin_r0_w,
                            att_src1, att_dst1)
    agg1 = agg_sc(fd, fs, cnts, as1.reshape(N), ad1.reshape(N), h)
    agg1 = agg1.reshape(NPAD, D)[:N]
    return _combine2(agg1, h, lin_l1_w, lin_l1_b, lin_r1_w, out_w, out_b)
